# baseline (speedup 1.0000x reference)
import functools

import jax
import jax.numpy as jnp
from jax import lax
from jax.experimental import pallas as pl
from jax.experimental.pallas import tpu as pltpu
from jax.experimental.pallas import tpu_sc as plsc

VOCAB = 1000000
EMBED_DIM = 32
PAD_DIM = 128
BATCH = 4096
HIST = 200

NC = 2
NS = 16
NW = NC * NS
B_ROWS = BATCH // NW
ROW_CHUNKS = ((0, 128), (128, 72))
ROWS_PER_SUPER = 4
SUPER = ROWS_PER_SUPER * HIST
N_SUPER = B_ROWS // ROWS_PER_SUPER
N_PAIR = N_SUPER // 2


TI = 16384
TGRID = -(-VOCAB // TI)


def _transpose_body(t_ref, o_ref):
  blk = t_ref[...]
  r = lax.broadcasted_iota(jnp.int32, (EMBED_DIM, EMBED_DIM), 0)
  c = lax.broadcasted_iota(jnp.int32, (EMBED_DIM, EMBED_DIM), 1)
  eye = jnp.where(r == c, 1.0, 0.0).astype(jnp.float32)
  t1 = lax.dot_general(blk, eye, (((0,), (0,)), ((), ())),
                       preferred_element_type=jnp.float32,
                       precision=lax.Precision.HIGHEST)
  t2 = t1.reshape(TI // 4, 4, EMBED_DIM)
  o_ref[...] = jnp.concatenate(
      [t2[:, q, :] for q in range(4)], axis=1)


_table_rm = pl.pallas_call(
    _transpose_body,
    grid=(TGRID,),
    in_specs=[pl.BlockSpec((EMBED_DIM, TI), lambda i: (0, i))],
    out_specs=pl.BlockSpec((TI // 4, 4 * EMBED_DIM), lambda i: (i, 0)),
    out_shape=jax.ShapeDtypeStruct((VOCAB // 4, 4 * EMBED_DIM), jnp.float32),
)


def _make_gather():
  mesh = plsc.VectorSubcoreMesh(core_axis_name="c", subcore_axis_name="s")

  @functools.partial(
      pl.kernel,
      mesh=mesh,
      out_type=jax.ShapeDtypeStruct((BATCH * HIST, PAD_DIM), jnp.float32),
      scratch_types=[
          pltpu.VMEM((B_ROWS, HIST), jnp.int32),
          pltpu.VMEM((2, SUPER, EMBED_DIM), jnp.float32),
          pltpu.SemaphoreType.DMA,
          pltpu.SemaphoreType.DMA,
          pltpu.SemaphoreType.DMA,
      ],
      compiler_params=pltpu.CompilerParams(use_tc_tiling_on_sc=False),
  )
  def gather_kernel(idx_hbm, table_hbm, out_hbm, idx_v, rows_v, gsem,
                    wsem0, wsem1):
    wid = lax.axis_index("s") * NC + lax.axis_index("c")
    base = wid * B_ROWS
    obase = base * HIST
    wsems = (wsem0, wsem1)

    pltpu.sync_copy(idx_hbm.at[pl.ds(base, B_ROWS)], idx_v)

    for s in range(2):
      pltpu.async_copy(
          rows_v.at[s],
          out_hbm.at[pl.ds(obase + s * SUPER, SUPER), pl.ds(0, EMBED_DIM)],
          wsems[s])

    def pair_body(p, carry):
      for s in range(2):
        g = 2 * p + s
        row0 = g * ROWS_PER_SUPER
        dst = out_hbm.at[pl.ds(obase + g * SUPER, SUPER),
                         pl.ds(0, EMBED_DIM)]
        pltpu.make_async_copy(rows_v.at[s], dst, wsems[s]).wait()
        for j in range(ROWS_PER_SUPER):
          for h, w in ROW_CHUNKS:
            pltpu.async_copy(
                table_hbm.at[idx_v.at[row0 + j, pl.ds(h, w)]],
                rows_v.at[s, pl.ds(j * HIST + h, w)], gsem)
        for j in range(ROWS_PER_SUPER):
          for h, w in ROW_CHUNKS:
            pltpu.make_async_copy(
                table_hbm.at[idx_v.at[row0 + j, pl.ds(h, w)]],
                rows_v.at[s, pl.ds(j * HIST + h, w)], gsem).wait()
        pltpu.async_copy(rows_v.at[s], dst, wsems[s])
      return carry

    lax.fori_loop(0, N_PAIR, pair_body, 0)

    for s in range(2):
      pltpu.make_async_copy(
          rows_v.at[s],
          out_hbm.at[pl.ds(obase + s * SUPER, SUPER), pl.ds(0, EMBED_DIM)],
          wsems[s]).wait()

  return gather_kernel


_gather = _make_gather()


def kernel(x, table):
  t_rm = _table_rm(table.T).reshape(VOCAB, EMBED_DIM)
  out_pad = _gather(x.astype(jnp.int32), t_rm)
  return out_pad[:, :EMBED_DIM].reshape(BATCH, HIST, EMBED_DIM)


# speedup vs baseline: 1.4170x; 1.4170x over previous
import functools

import jax
import jax.numpy as jnp
from jax import lax
from jax.experimental import pallas as pl
from jax.experimental.pallas import tpu as pltpu
from jax.experimental.pallas import tpu_sc as plsc

VOCAB = 1000000
EMBED_DIM = 32
PAD_DIM = 128
BATCH = 4096
HIST = 200

NC = 2
NS = 16
NW = NC * NS
B_ROWS = BATCH // NW
ROW_CHUNKS = ((0, 128), (128, 72))
ROWS_PER_SUPER = 4
SUPER = ROWS_PER_SUPER * HIST
N_SUPER = B_ROWS // ROWS_PER_SUPER
N_PAIR = N_SUPER // 2


TI = 8192
TGRID = -(-VOCAB // TI)


def _transpose_body(t_ref, o_ref):
  blk = t_ref[...]
  t1 = blk.T
  t2 = t1.reshape(TI // 4, 4, EMBED_DIM)
  o_ref[...] = jnp.concatenate(
      [t2[:, q, :] for q in range(4)], axis=1)


_table_rm = pl.pallas_call(
    _transpose_body,
    grid=(TGRID,),
    in_specs=[pl.BlockSpec((EMBED_DIM, TI), lambda i: (0, i))],
    out_specs=pl.BlockSpec((TI // 4, 4 * EMBED_DIM), lambda i: (i, 0)),
    out_shape=jax.ShapeDtypeStruct((VOCAB // 4, 4 * EMBED_DIM), jnp.float32),
)


def _make_gather():
  mesh = plsc.VectorSubcoreMesh(core_axis_name="c", subcore_axis_name="s")

  @functools.partial(
      pl.kernel,
      mesh=mesh,
      out_type=jax.ShapeDtypeStruct((BATCH * HIST, PAD_DIM), jnp.float32),
      scratch_types=[
          pltpu.VMEM((B_ROWS, HIST), jnp.int32),
          pltpu.VMEM((2, SUPER, EMBED_DIM), jnp.float32),
          pltpu.SemaphoreType.DMA,
          pltpu.SemaphoreType.DMA,
          pltpu.SemaphoreType.DMA,
      ],
      compiler_params=pltpu.CompilerParams(use_tc_tiling_on_sc=False),
  )
  def gather_kernel(idx_hbm, table_hbm, out_hbm, idx_v, rows_v, gsem,
                    wsem0, wsem1):
    wid = lax.axis_index("s") * NC + lax.axis_index("c")
    base = wid * B_ROWS
    obase = base * HIST
    wsems = (wsem0, wsem1)

    pltpu.sync_copy(idx_hbm.at[pl.ds(base, B_ROWS)], idx_v)

    for s in range(2):
      pltpu.async_copy(
          rows_v.at[s],
          out_hbm.at[pl.ds(obase + s * SUPER, SUPER), pl.ds(0, EMBED_DIM)],
          wsems[s])

    def pair_body(p, carry):
      for s in range(2):
        g = 2 * p + s
        row0 = g * ROWS_PER_SUPER
        dst = out_hbm.at[pl.ds(obase + g * SUPER, SUPER),
                         pl.ds(0, EMBED_DIM)]
        pltpu.make_async_copy(rows_v.at[s], dst, wsems[s]).wait()
        for j in range(ROWS_PER_SUPER):
          for h, w in ROW_CHUNKS:
            pltpu.async_copy(
                table_hbm.at[idx_v.at[row0 + j, pl.ds(h, w)]],
                rows_v.at[s, pl.ds(j * HIST + h, w)], gsem)
        for j in range(ROWS_PER_SUPER):
          for h, w in ROW_CHUNKS:
            pltpu.make_async_copy(
                table_hbm.at[idx_v.at[row0 + j, pl.ds(h, w)]],
                rows_v.at[s, pl.ds(j * HIST + h, w)], gsem).wait()
        pltpu.async_copy(rows_v.at[s], dst, wsems[s])
      return carry

    lax.fori_loop(0, N_PAIR, pair_body, 0)

    for s in range(2):
      pltpu.make_async_copy(
          rows_v.at[s],
          out_hbm.at[pl.ds(obase + s * SUPER, SUPER), pl.ds(0, EMBED_DIM)],
          wsems[s]).wait()

  return gather_kernel


_gather = _make_gather()


def kernel(x, table):
  t_rm = _table_rm(table.T).reshape(VOCAB, EMBED_DIM)
  out_pad = _gather(x.astype(jnp.int32), t_rm)
  return out_pad[:, :EMBED_DIM].reshape(BATCH, HIST, EMBED_DIM)


# speedup vs baseline: 1.4392x; 1.0157x over previous
import functools

import jax
import jax.numpy as jnp
from jax import lax
from jax.experimental import pallas as pl
from jax.experimental.pallas import tpu as pltpu
from jax.experimental.pallas import tpu_sc as plsc

VOCAB = 1000000
EMBED_DIM = 32
PAD_DIM = 128
BATCH = 4096
HIST = 200

NC = 2
NS = 16
NW = NC * NS
B_ROWS = BATCH // NW
ROW_CHUNKS = ((0, 128), (128, 72))
ROWS_PER_SUPER = 4
SUPER = ROWS_PER_SUPER * HIST
N_SUPER = B_ROWS // ROWS_PER_SUPER
N_PAIR = N_SUPER // 2


TI = 32768
TGRID = -(-VOCAB // TI)


def _transpose_body(t_ref, o_ref):
  blk = t_ref[...]
  t1 = blk.T
  t2 = t1.reshape(TI // 4, 4, EMBED_DIM)
  o_ref[...] = jnp.concatenate(
      [t2[:, q, :] for q in range(4)], axis=1)


_table_rm = pl.pallas_call(
    _transpose_body,
    grid=(TGRID,),
    in_specs=[pl.BlockSpec((EMBED_DIM, TI), lambda i: (0, i))],
    out_specs=pl.BlockSpec((TI // 4, 4 * EMBED_DIM), lambda i: (i, 0)),
    out_shape=jax.ShapeDtypeStruct((VOCAB // 4, 4 * EMBED_DIM), jnp.float32),
)


def _make_gather():
  mesh = plsc.VectorSubcoreMesh(core_axis_name="c", subcore_axis_name="s")

  @functools.partial(
      pl.kernel,
      mesh=mesh,
      out_type=jax.ShapeDtypeStruct((BATCH * HIST, PAD_DIM), jnp.float32),
      scratch_types=[
          pltpu.VMEM((B_ROWS, HIST), jnp.int32),
          pltpu.VMEM((2, SUPER, EMBED_DIM), jnp.float32),
          pltpu.SemaphoreType.DMA,
          pltpu.SemaphoreType.DMA,
          pltpu.SemaphoreType.DMA,
      ],
      compiler_params=pltpu.CompilerParams(use_tc_tiling_on_sc=False),
  )
  def gather_kernel(idx_hbm, table_hbm, out_hbm, idx_v, rows_v, gsem,
                    wsem0, wsem1):
    wid = lax.axis_index("s") * NC + lax.axis_index("c")
    base = wid * B_ROWS
    obase = base * HIST
    wsems = (wsem0, wsem1)

    pltpu.sync_copy(idx_hbm.at[pl.ds(base, B_ROWS)], idx_v)

    for s in range(2):
      pltpu.async_copy(
          rows_v.at[s],
          out_hbm.at[pl.ds(obase + s * SUPER, SUPER), pl.ds(0, EMBED_DIM)],
          wsems[s])

    def pair_body(p, carry):
      for s in range(2):
        g = 2 * p + s
        row0 = g * ROWS_PER_SUPER
        dst = out_hbm.at[pl.ds(obase + g * SUPER, SUPER),
                         pl.ds(0, EMBED_DIM)]
        pltpu.make_async_copy(rows_v.at[s], dst, wsems[s]).wait()
        for j in range(ROWS_PER_SUPER):
          for h, w in ROW_CHUNKS:
            pltpu.async_copy(
                table_hbm.at[idx_v.at[row0 + j, pl.ds(h, w)]],
                rows_v.at[s, pl.ds(j * HIST + h, w)], gsem)
        for j in range(ROWS_PER_SUPER):
          for h, w in ROW_CHUNKS:
            pltpu.make_async_copy(
                table_hbm.at[idx_v.at[row0 + j, pl.ds(h, w)]],
                rows_v.at[s, pl.ds(j * HIST + h, w)], gsem).wait()
        pltpu.async_copy(rows_v.at[s], dst, wsems[s])
      return carry

    lax.fori_loop(0, N_PAIR, pair_body, 0)

    for s in range(2):
      pltpu.make_async_copy(
          rows_v.at[s],
          out_hbm.at[pl.ds(obase + s * SUPER, SUPER), pl.ds(0, EMBED_DIM)],
          wsems[s]).wait()

  return gather_kernel


_gather = _make_gather()


def kernel(x, table):
  t_rm = _table_rm(table.T).reshape(VOCAB, EMBED_DIM)
  out_pad = _gather(x.astype(jnp.int32), t_rm)
  return out_pad[:, :EMBED_DIM].reshape(BATCH, HIST, EMBED_DIM)
